# baseline (speedup 1.0000x reference)
import numpy as np
import jax
import jax.numpy as jnp
from jax.experimental import pallas as pl

IOU_THR = 0.5
TOP_K = 10
NPF = 128

_S = (9.999972890160e-01, -1.666514581916e-01, 8.319841916372e-03,
      -1.942414723508e-04, 2.224867303557e-06)
_INV2PI = 0.15915494309189535
_C1 = np.float32(2.0 * np.pi)
_C2 = np.float32(2.0 * np.pi - np.float64(_C1))


_LN2 = 0.6931471805599453
_SQRT2 = 1.4142135623730951


def _fast_log(d):
    bits = jax.lax.bitcast_convert_type(d, jnp.int32)
    e = jnp.right_shift(bits, 23) - 127
    m = jax.lax.bitcast_convert_type(
        jnp.bitwise_or(jnp.bitwise_and(bits, 0x007FFFFF), 0x3F800000),
        jnp.float32)
    c = m > _SQRT2
    m2 = jnp.where(c, 0.5 * m, m)
    ef = e.astype(jnp.float32) + jnp.where(c, 1.0, 0.0)
    t = (m2 - 1.0) / (m2 + 1.0)
    t2 = t * t
    p = 2.0 / 7.0
    for coef in (2.0 / 5.0, 2.0 / 3.0, 2.0):
        p = p * t2 + coef
    return ef * _LN2 + t * p


def _fast_sin(y):
    k = jnp.round(y * _INV2PI)
    r = y - k * _C1 - k * _C2
    r2 = r * r
    p = _S[4]
    for i in (3, 2, 1, 0):
        p = p * r2 + _S[i]
    return r * p


def _body(boxes_i_ref, g48_ref, boxesT_ref, seed_col_ref,
          seed_row_ref, tgt_ref, w1t_ref, b1_ref, w2t_ref, b2_ref,
          w3wt_ref, s3_ref, b3_ref, w4t_ref, b4_ref, w5t_ref, b5_ref,
          ang4_ref, shift_ref, out_ref, mask_ref):
    f32 = jnp.float32
    T = boxes_i_ref.shape[1]
    N = boxesT_ref.shape[2]
    D = tgt_ref.shape[2]

    bi = boxes_i_ref[0]
    bT = boxesT_ref[0]

    cx_i, cy_i, w_i, h_i = bi[:, 0:1], bi[:, 1:2], bi[:, 2:3], bi[:, 3:4]
    cx_j, cy_j, w_j, h_j = bT[0:1, :], bT[1:2, :], bT[2:3, :], bT[3:4, :]

    x0_i = cx_i - 0.5 * w_i
    x1_i = cx_i + 0.5 * w_i
    y0_i = cy_i - 0.5 * h_i
    y1_i = cy_i + 0.5 * h_i
    x0_j = cx_j - 0.5 * w_j
    x1_j = cx_j + 0.5 * w_j
    y0_j = cy_j - 0.5 * h_j
    y1_j = cy_j + 0.5 * h_j

    iw = jnp.maximum(jnp.minimum(x1_i, x1_j) - jnp.maximum(x0_i, x0_j), 0.0)
    ih = jnp.maximum(jnp.minimum(y1_i, y1_j) - jnp.maximum(y0_i, y0_j), 0.0)
    inter = iw * ih
    area_i = (x1_i - x0_i) * (y1_i - y0_i)
    area_j = (x1_j - x0_j) * (y1_j - y0_j)
    union = area_i + area_j - inter
    iou = inter / jnp.maximum(union, 1e-9)

    mask_ref[0] = iou >= IOU_THR

    neg_col = 1.0 - seed_col_ref[0]
    ov = iou * seed_row_ref[0] * neg_col

    iota = jax.lax.broadcasted_iota(jnp.int32, (T, N), 1)
    iota_s = jax.lax.broadcasted_iota(jnp.int32, (T, 256), 1)
    g48 = g48_ref[0]

    macc = jnp.full((T, D), -jnp.inf, dtype=f32)
    for _ in range(TOP_K):
        m = jnp.max(ov, axis=1, keepdims=True)
        idx = jnp.min(jnp.where(ov == m, iota, N), axis=1, keepdims=True)
        onehot = iota == idx
        r = jnp.bitwise_and(idx, 255)
        q = jnp.right_shift(idx, 8)
        oh = (iota_s == r).astype(jnp.bfloat16)
        g3 = jnp.dot(oh, g48, preferred_element_type=f32)
        nb4 = g3[:, 0:16] + g3[:, 16:32] + g3[:, 32:48]
        nb = sum((q == g).astype(f32) * nb4[:, 4 * g:4 * g + 4]
                 for g in range(4))
        ov = jnp.where(onehot, -1.0, ov)
        mk = (m >= IOU_THR).astype(f32)
        v = m * mk
        x = _fast_log(jnp.maximum(jnp.abs(nb - bi), 1e-7))
        xb = jnp.concatenate(
            [jnp.broadcast_to(x[:, c:c + 1], (T, 64)) for c in range(4)],
            axis=1)
        waves = _fast_sin(xb * ang4_ref[...] + shift_ref[...])
        h = jnp.maximum(
            jnp.dot(waves, w3wt_ref[...], preferred_element_type=f32)
            + v * s3_ref[...] + b3_ref[...], 0.0)
        fk = jnp.dot(h, w4t_ref[...], preferred_element_type=f32) + b4_ref[...]
        macc = jnp.maximum(macc, fk * mk)

    h1 = jnp.maximum(
        jnp.dot(tgt_ref[0], w1t_ref[...], preferred_element_type=f32)
        + b1_ref[...], 0.0)
    cur = jnp.dot(h1, w2t_ref[...], preferred_element_type=f32) + b2_ref[...]
    pre = cur * neg_col + macc
    out = jnp.maximum(
        jnp.dot(pre, w5t_ref[...], preferred_element_type=f32)
        + b5_ref[...], 0.0) * neg_col
    out_ref[0] = out


def kernel(tgt, seed_mask, pred_boxes, W1, b1, W2, b2, W3, b3, W4, b4, W5, b5):
    bs, N, D = tgt.shape
    T = 1000

    boxesT = jnp.transpose(pred_boxes, (0, 2, 1))
    seed_row = jnp.transpose(seed_mask, (0, 2, 1))

    f32 = jnp.float32
    bp = jnp.pad(pred_boxes, ((0, 0), (0, 1024 - N), (0, 0)))
    grouped = jnp.transpose(bp.reshape(bs, 4, 256, 4),
                            (0, 2, 1, 3)).reshape(bs, 256, 16)
    g_hi = grouped.astype(jnp.bfloat16)
    r1 = grouped - g_hi.astype(f32)
    g_mid = r1.astype(jnp.bfloat16)
    g_lo = (r1 - g_mid.astype(f32)).astype(jnp.bfloat16)
    g48 = jnp.concatenate([g_hi, g_mid, g_lo], axis=-1)

    s3 = jnp.sum(W3[:, :64], axis=1)[None, :]
    w3w = W3[:, 64:].reshape(D, 2, 2, NPF // 2, 2).sum(-1).reshape(D, 256)
    w3wt = jnp.transpose(w3w)

    dim_t2 = 10000.0 ** (2.0 * np.arange(NPF // 2) / NPF)
    ang1 = (2.0 * np.pi) / dim_t2
    ang4 = jnp.asarray(np.tile(ang1, 4), jnp.float32)[None, :]
    shift = jnp.asarray(
        np.concatenate([np.zeros(64), np.full(64, 0.5 * np.pi)] * 2),
        jnp.float32)[None, :]

    row = lambda b: b[None, :]

    def const(shape):
        return pl.BlockSpec(shape, lambda b, i: (0,) * len(shape))

    out, mask = pl.pallas_call(
        _body,
        grid=(bs, N // T),
        in_specs=[
            pl.BlockSpec((1, T, 4), lambda b, i: (b, i, 0)),
            pl.BlockSpec((1, 256, 48), lambda b, i: (b, 0, 0)),
            pl.BlockSpec((1, 4, N), lambda b, i: (b, 0, 0)),
            pl.BlockSpec((1, T, 1), lambda b, i: (b, i, 0)),
            pl.BlockSpec((1, 1, N), lambda b, i: (b, 0, 0)),
            pl.BlockSpec((1, T, D), lambda b, i: (b, i, 0)),
            const((D, D)), const((1, D)),
            const((D, D)), const((1, D)),
            const((256, D)), const((1, D)), const((1, D)),
            const((D, D)), const((1, D)),
            const((D, D)), const((1, D)),
            const((1, 256)), const((1, 256)),
        ],
        out_specs=[
            pl.BlockSpec((1, T, D), lambda b, i: (b, i, 0)),
            pl.BlockSpec((1, T, N), lambda b, i: (b, i, 0)),
        ],
        out_shape=[
            jax.ShapeDtypeStruct((bs, N, D), jnp.float32),
            jax.ShapeDtypeStruct((bs, N, N), jnp.bool_),
        ],
    )(pred_boxes, g48, boxesT, seed_mask, seed_row, tgt,
      W1.T, row(b1), W2.T, row(b2), w3wt, s3, row(b3),
      W4.T, row(b4), W5.T, row(b5), ang4, shift)
    return out, mask


# speedup vs baseline: 1.3469x; 1.3469x over previous
import numpy as np
import jax
import jax.numpy as jnp
from jax.experimental import pallas as pl

IOU_THR = 0.5
TOP_K = 10
NPF = 128

_S = (9.999972890160e-01, -1.666514581916e-01, 8.319841916372e-03,
      -1.942414723508e-04, 2.224867303557e-06)
_INV2PI = 0.15915494309189535
_C1 = np.float32(2.0 * np.pi)
_C2 = np.float32(2.0 * np.pi - np.float64(_C1))


def _fast_sin(y):
    k = jnp.round(y * _INV2PI)
    r = y - k * _C1 - k * _C2
    r2 = r * r
    p = _S[4]
    for i in (3, 2, 1, 0):
        p = p * r2 + _S[i]
    return r * p


def _body(boxes_i_ref, g48_ref, boxesT_ref, seed_col_ref,
          seed_row_ref, tgt_ref, w1t_ref, b1_ref, w2t_ref, b2_ref,
          w3wt_ref, s3_ref, b3_ref, w4t_ref, b4_ref, w5t_ref, b5_ref,
          ang4_ref, shift_ref, out_ref, mask_ref):
    f32 = jnp.float32
    T = boxes_i_ref.shape[1]
    N = boxesT_ref.shape[2]
    D = tgt_ref.shape[2]

    bi = boxes_i_ref[0]
    bT = boxesT_ref[0]

    cx_i, cy_i, w_i, h_i = bi[:, 0:1], bi[:, 1:2], bi[:, 2:3], bi[:, 3:4]
    cx_j, cy_j, w_j, h_j = bT[0:1, :], bT[1:2, :], bT[2:3, :], bT[3:4, :]

    x0_i = cx_i - 0.5 * w_i
    x1_i = cx_i + 0.5 * w_i
    y0_i = cy_i - 0.5 * h_i
    y1_i = cy_i + 0.5 * h_i
    x0_j = cx_j - 0.5 * w_j
    x1_j = cx_j + 0.5 * w_j
    y0_j = cy_j - 0.5 * h_j
    y1_j = cy_j + 0.5 * h_j

    iw = jnp.maximum(jnp.minimum(x1_i, x1_j) - jnp.maximum(x0_i, x0_j), 0.0)
    ih = jnp.maximum(jnp.minimum(y1_i, y1_j) - jnp.maximum(y0_i, y0_j), 0.0)
    inter = iw * ih
    area_i = (x1_i - x0_i) * (y1_i - y0_i)
    area_j = (x1_j - x0_j) * (y1_j - y0_j)
    union = area_i + area_j - inter
    iou = inter / jnp.maximum(union, 1e-9)

    mask_ref[0] = iou >= IOU_THR

    neg_col = 1.0 - seed_col_ref[0]
    ov = iou * seed_row_ref[0] * neg_col

    iota = jax.lax.broadcasted_iota(jnp.int32, (T, N), 1)
    iota_s = jax.lax.broadcasted_iota(jnp.int32, (T, 256), 1)
    g48 = g48_ref[0]

    nbs, ms = [], []
    for _ in range(TOP_K):
        m = jnp.max(ov, axis=1, keepdims=True)
        idx = jnp.min(jnp.where(ov == m, iota, N), axis=1, keepdims=True)
        onehot = iota == idx
        r = jnp.bitwise_and(idx, 255)
        q = jnp.right_shift(idx, 8)
        oh = (iota_s == r).astype(jnp.bfloat16)
        g3 = jnp.dot(oh, g48, preferred_element_type=f32)
        nb4 = g3[:, 0:16] + g3[:, 16:32] + g3[:, 32:48]
        nb = sum((q == g).astype(f32) * nb4[:, 4 * g:4 * g + 4]
                 for g in range(4))
        ov = jnp.where(onehot, -1.0, ov)
        nbs.append(nb)
        ms.append(m)

    nb_all = jnp.concatenate(nbs, axis=1)
    bi_all = jnp.concatenate([bi] * TOP_K, axis=1)
    x_all = jnp.log(jnp.maximum(jnp.abs(nb_all - bi_all), 1e-7))

    macc = jnp.full((T, D), -jnp.inf, dtype=f32)
    for k in range(TOP_K):
        m = ms[k]
        mk = (m >= IOU_THR).astype(f32)
        v = m * mk
        xb = jnp.concatenate(
            [jnp.broadcast_to(x_all[:, 4 * k + c:4 * k + c + 1], (T, 64))
             for c in range(4)], axis=1)
        waves = _fast_sin(xb * ang4_ref[...] + shift_ref[...])
        h = jnp.maximum(
            jnp.dot(waves, w3wt_ref[...], preferred_element_type=f32)
            + v * s3_ref[...] + b3_ref[...], 0.0)
        fk = jnp.dot(h, w4t_ref[...], preferred_element_type=f32) + b4_ref[...]
        macc = jnp.maximum(macc, fk * mk)

    h1 = jnp.maximum(
        jnp.dot(tgt_ref[0], w1t_ref[...], preferred_element_type=f32)
        + b1_ref[...], 0.0)
    cur = jnp.dot(h1, w2t_ref[...], preferred_element_type=f32) + b2_ref[...]
    pre = cur * neg_col + macc
    out = jnp.maximum(
        jnp.dot(pre, w5t_ref[...], preferred_element_type=f32)
        + b5_ref[...], 0.0) * neg_col
    out_ref[0] = out


def kernel(tgt, seed_mask, pred_boxes, W1, b1, W2, b2, W3, b3, W4, b4, W5, b5):
    bs, N, D = tgt.shape
    T = 1000

    boxesT = jnp.transpose(pred_boxes, (0, 2, 1))
    seed_row = jnp.transpose(seed_mask, (0, 2, 1))

    f32 = jnp.float32
    bp = jnp.pad(pred_boxes, ((0, 0), (0, 1024 - N), (0, 0)))
    grouped = jnp.transpose(bp.reshape(bs, 4, 256, 4),
                            (0, 2, 1, 3)).reshape(bs, 256, 16)
    g_hi = grouped.astype(jnp.bfloat16)
    r1 = grouped - g_hi.astype(f32)
    g_mid = r1.astype(jnp.bfloat16)
    g_lo = (r1 - g_mid.astype(f32)).astype(jnp.bfloat16)
    g48 = jnp.concatenate([g_hi, g_mid, g_lo], axis=-1)

    s3 = jnp.sum(W3[:, :64], axis=1)[None, :]
    w3w = W3[:, 64:].reshape(D, 2, 2, NPF // 2, 2).sum(-1).reshape(D, 256)
    w3wt = jnp.transpose(w3w)

    dim_t2 = 10000.0 ** (2.0 * np.arange(NPF // 2) / NPF)
    ang1 = (2.0 * np.pi) / dim_t2
    ang4 = jnp.asarray(np.tile(ang1, 4), jnp.float32)[None, :]
    shift = jnp.asarray(
        np.concatenate([np.zeros(64), np.full(64, 0.5 * np.pi)] * 2),
        jnp.float32)[None, :]

    row = lambda b: b[None, :]

    def const(shape):
        return pl.BlockSpec(shape, lambda b, i: (0,) * len(shape))

    out, mask = pl.pallas_call(
        _body,
        grid=(bs, N // T),
        in_specs=[
            pl.BlockSpec((1, T, 4), lambda b, i: (b, i, 0)),
            pl.BlockSpec((1, 256, 48), lambda b, i: (b, 0, 0)),
            pl.BlockSpec((1, 4, N), lambda b, i: (b, 0, 0)),
            pl.BlockSpec((1, T, 1), lambda b, i: (b, i, 0)),
            pl.BlockSpec((1, 1, N), lambda b, i: (b, 0, 0)),
            pl.BlockSpec((1, T, D), lambda b, i: (b, i, 0)),
            const((D, D)), const((1, D)),
            const((D, D)), const((1, D)),
            const((256, D)), const((1, D)), const((1, D)),
            const((D, D)), const((1, D)),
            const((D, D)), const((1, D)),
            const((1, 256)), const((1, 256)),
        ],
        out_specs=[
            pl.BlockSpec((1, T, D), lambda b, i: (b, i, 0)),
            pl.BlockSpec((1, T, N), lambda b, i: (b, i, 0)),
        ],
        out_shape=[
            jax.ShapeDtypeStruct((bs, N, D), jnp.float32),
            jax.ShapeDtypeStruct((bs, N, N), jnp.bool_),
        ],
    )(pred_boxes, g48, boxesT, seed_mask, seed_row, tgt,
      W1.T, row(b1), W2.T, row(b2), w3wt, s3, row(b3),
      W4.T, row(b4), W5.T, row(b5), ang4, shift)
    return out, mask
